# baseline (speedup 1.0000x reference)
import functools

import jax
import jax.numpy as jnp
from jax import lax
from jax.experimental import pallas as pl
from jax.experimental.pallas import tpu as pltpu
from jax.experimental.pallas import tpu_sc as plsc

B, N, D = 1024, 100000, 1024
ES_DIM = 101
TOP_K = 90
TEMP = 0.04
BN = 2048
NBLK = 49
NP = NBLK * BN
CAP = 2048
CAPV = CAP // 16
NVREG = NP // 16
NW = 32
QPW = B // NW
NLEV = 8
NEG = float("-inf")


def _qnorm_body(x_ref, o_ref):
    x = x_ref[:]
    ss = jnp.sum(x * x, axis=1, keepdims=True)
    n = jnp.maximum(jnp.sqrt(ss), 1e-12)
    o_ref[:] = (x / n).astype(jnp.bfloat16)


def _p0(en_1024):
    return pl.pallas_call(
        _qnorm_body,
        out_shape=jax.ShapeDtypeStruct((B, D), jnp.bfloat16),
    )(en_1024)


def _sims_body(qn_ref, db_ref, sims_ref, stats_ref, acc_ref):
    j = pl.program_id(0)
    db = db_ref[:]
    ss = jnp.sum(db * db, axis=1, keepdims=True)
    dbn = (db / jnp.maximum(jnp.sqrt(ss), 1e-12)).astype(jnp.bfloat16)
    s = jax.lax.dot_general(qn_ref[:], dbn, (((1,), (1,)), ((), ())),
                            preferred_element_type=jnp.float32)
    col = j * BN + jax.lax.broadcasted_iota(jnp.int32, (B, BN), 1)
    valid = col < N
    s_m = jnp.where(valid, s, NEG)
    sims_ref[:] = s_m
    bmax = jnp.max(s_m, axis=1, keepdims=True)
    bsum = jnp.sum(jnp.where(valid, s, 0.0), axis=1, keepdims=True)
    bmin = jnp.min(jnp.where(valid, s, jnp.inf), axis=1, keepdims=True)
    first = j == 0
    a = acc_ref[:]
    nmax = jnp.where(first, bmax, jnp.maximum(a[:, 0:1], bmax))
    nsum = jnp.where(first, bsum, a[:, 1:2] + bsum)
    nmin = jnp.where(first, bmin, jnp.minimum(a[:, 2:3], bmin))
    acc_ref[:] = jnp.concatenate(
        [nmax, nsum, nmin, jnp.zeros((B, 5), jnp.float32)], axis=1)

    @pl.when(j == NBLK - 1)
    def _():
        stats_ref[:] = acc_ref[:]


def _p1(qn, en_db):
    return pl.pallas_call(
        _sims_body,
        grid=(NBLK,),
        in_specs=[
            pl.BlockSpec((B, D), lambda j: (0, 0)),
            pl.BlockSpec((BN, D), lambda j: (j, 0)),
        ],
        out_specs=[
            pl.BlockSpec((B, BN), lambda j: (0, j)),
            pl.BlockSpec((B, 8), lambda j: (0, 0)),
        ],
        out_shape=[
            jax.ShapeDtypeStruct((B, NP), jnp.float32),
            jax.ShapeDtypeStruct((B, 8), jnp.float32),
        ],
        scratch_shapes=[pltpu.VMEM((B, 8), jnp.float32)],
    )(qn, en_db)


def _count_body(sims_ref, stats_ref, lo_ref, cnt_ref):
    j = pl.program_id(0)
    st = stats_ref[:]
    mx = st[:, 0:1]
    mean = st[:, 1:2] * (1.0 / N)
    mn = st[:, 2:3]
    lane = jax.lax.broadcasted_iota(jnp.int32, (B, NLEV), 1)
    frac = (lane.astype(jnp.float32) + 1.0) * (1.0 / (NLEV - 1.0))
    levels = mx - (mx - mean) * frac
    levels = jnp.where(lane == NLEV - 1, mn, levels)
    s = sims_ref[:]
    parts = []
    for l in range(NLEV):
        m = (s >= levels[:, l:l + 1]).astype(jnp.float32)
        parts.append(jnp.sum(m, axis=1, keepdims=True))
    blk = jnp.concatenate(parts, axis=1)
    cnt_ref[:] = jnp.where(j == 0, blk, cnt_ref[:] + blk)

    @pl.when(j == NBLK - 1)
    def _():
        cnt = cnt_ref[:]
        lo = jnp.max(jnp.where(cnt >= TOP_K, levels, NEG), axis=1, keepdims=True)
        lo_ref[:] = jnp.broadcast_to(lo, (B, 8))


def _p2(sims, stats):
    return pl.pallas_call(
        _count_body,
        grid=(NBLK,),
        in_specs=[
            pl.BlockSpec((B, BN), lambda j: (0, j)),
            pl.BlockSpec((B, 8), lambda j: (0, 0)),
        ],
        out_specs=pl.BlockSpec((B, 8), lambda j: (0, 0)),
        out_shape=jax.ShapeDtypeStruct((B, 8), jnp.float32),
        scratch_shapes=[pltpu.VMEM((B, NLEV), jnp.float32)],
    )(sims, stats)


NCH = 4
CH = NP // NCH
CHV = CH // 16


def _sc_compact_body(sims_hbm, lo_hbm, cand_hbm, buf0, buf1, cand_v, lo_v,
                     sem0, sem1):
    wid = lax.axis_index("s") * 2 + lax.axis_index("c")
    base = wid * QPW
    pltpu.sync_copy(lo_hbm.at[pl.ds(base, QPW)], lo_v.at[pl.ds(0, QPW)])
    zeros16 = jnp.zeros((16,), jnp.int32)
    neg16 = jnp.full((16,), NEG, jnp.float32)
    bufs = (buf0, buf1)
    sems = (sem0, sem1)

    def per_query(k, carry):
        q = base + k
        lo_vec = jnp.zeros((16,), jnp.float32) + lo_v[pl.ds(k, 16)][0]

        @plsc.parallel_loop(0, CAPV, unroll=8)
        def clear(s):
            cand_v[pl.ds(s * 16, 16)] = neg16

        copies = [pltpu.async_copy(sims_hbm.at[q, pl.ds(0, CH)], bufs[0], sems[0])]
        off = jnp.int32(0)
        for ci in range(NCH):
            copies[ci].wait()
            if ci + 1 < NCH:
                copies.append(pltpu.async_copy(
                    sims_hbm.at[q, pl.ds((ci + 1) * CH, CH)],
                    bufs[(ci + 1) % 2], sems[(ci + 1) % 2]))
            buf = bufs[ci % 2]

            def pass1(i, off_s, buf=buf):
                v = buf[pl.ds(i * 16, 16)]
                m = v >= lo_vec
                ones = jnp.where(m, jnp.int32(1), jnp.int32(0))
                c = plsc.cumsum(ones)
                dest = jnp.minimum(zeros16 + off_s + c - 1, CAP - 1)
                plsc.store_scatter(cand_v, [dest], v, mask=m)
                return off_s + c[15]

            off = plsc.parallel_loop(0, CHV, carry=off, unroll=16)(pass1)

        pltpu.sync_copy(cand_v, cand_hbm.at[q])
        return carry

    lax.fori_loop(0, QPW, per_query, 0)


def _p3(sims, lo1):
    f = functools.partial(
        pl.kernel,
        out_type=jax.ShapeDtypeStruct((B, CAP), jnp.float32),
        mesh=plsc.VectorSubcoreMesh(core_axis_name="c", subcore_axis_name="s"),
        compiler_params=pltpu.CompilerParams(needs_layout_passes=False),
        scratch_types=[
            pltpu.VMEM((CH,), jnp.float32),
            pltpu.VMEM((CH,), jnp.float32),
            pltpu.VMEM((CAP,), jnp.float32),
            pltpu.VMEM((128,), jnp.float32),
            pltpu.SemaphoreType.DMA,
            pltpu.SemaphoreType.DMA,
        ],
    )(_sc_compact_body)
    return f(sims, lo1)


def _bisect_body(cand_ref, t_ref):
    c = cand_ref[:]
    bu = jax.lax.bitcast_convert_type(c, jnp.uint32)
    key = jnp.where(c >= 0, bu ^ jnp.uint32(0x80000000), ~bu)
    rows = c.shape[0]
    t = jnp.zeros((rows, 1), jnp.uint32)
    for bit in range(31, -1, -1):
        trial = t + jnp.uint32(1 << bit)
        n_ge = jnp.sum((key >= trial).astype(jnp.int32), axis=1, keepdims=True)
        t = jnp.where(n_ge >= TOP_K, trial, t)
    fb = jnp.where(t >= jnp.uint32(0x80000000), t ^ jnp.uint32(0x80000000), ~t)
    v90 = jax.lax.bitcast_convert_type(fb, jnp.float32)
    t_ref[:] = jnp.broadcast_to(v90, (rows, 8))


def _p4(cand):
    rb = 512
    return pl.pallas_call(
        _bisect_body,
        grid=(B // rb,),
        in_specs=[pl.BlockSpec((rb, CAP), lambda r: (r, 0))],
        out_specs=pl.BlockSpec((rb, 8), lambda r: (r, 0)),
        out_shape=jax.ShapeDtypeStruct((B, 8), jnp.float32),
    )(cand)


def _selmat_body(es_ref, p_ref, out_ref):
    j = pl.program_id(0)
    es_sp = jax.lax.dot_general(es_ref[:].astype(jnp.bfloat16), p_ref[:],
                                (((1,), (0,)), ((), ())),
                                preferred_element_type=jnp.float32)
    row = j * BN + jax.lax.broadcasted_iota(jnp.int32, (BN, 128), 0)
    colid = jax.lax.broadcasted_iota(jnp.int32, (BN, 128), 1)
    es_sp = jnp.where(colid == ES_DIM, 1.0, jnp.where(row < N, es_sp, 0.0))
    out_ref[:] = es_sp.astype(jnp.bfloat16)


def _p2b(es_db, p_bf):
    return pl.pallas_call(
        _selmat_body,
        grid=(NBLK,),
        in_specs=[
            pl.BlockSpec((BN, D), lambda j: (j, 0)),
            pl.BlockSpec((D, 128), lambda j: (0, 0)),
        ],
        out_specs=pl.BlockSpec((BN, 128), lambda j: (j, 0)),
        out_shape=jax.ShapeDtypeStruct((NP, 128), jnp.bfloat16),
    )(es_db, p_bf)


def _combine_body(sims_ref, esp_ref, t_ref, stats_ref, out_ref, acc_ref):
    j = pl.program_id(0)
    s = sims_ref[:]
    t = t_ref[:, 0:1]
    mx = stats_ref[:, 0:1]
    w = jnp.where(s >= t, jnp.exp((s - mx) * (1.0 / TEMP)), 0.0)
    contrib = jax.lax.dot_general(w.astype(jnp.bfloat16), esp_ref[:],
                                  (((1,), (0,)), ((), ())),
                                  preferred_element_type=jnp.float32)
    acc_ref[:] = jnp.where(j == 0, contrib, acc_ref[:] + contrib)

    @pl.when(j == NBLK - 1)
    def _():
        a = acc_ref[:]
        out_ref[:] = a / a[:, ES_DIM:ES_DIM + 1]


def _p5(sims, es_spb, t, stats):
    return pl.pallas_call(
        _combine_body,
        grid=(NBLK,),
        in_specs=[
            pl.BlockSpec((B, BN), lambda j: (0, j)),
            pl.BlockSpec((BN, 128), lambda j: (j, 0)),
            pl.BlockSpec((B, 8), lambda j: (0, 0)),
            pl.BlockSpec((B, 8), lambda j: (0, 0)),
        ],
        out_specs=pl.BlockSpec((B, 128), lambda j: (0, 0)),
        out_shape=jax.ShapeDtypeStruct((B, 128), jnp.float32),
        scratch_shapes=[pltpu.VMEM((B, 128), jnp.float32)],
    )(sims, es_spb, t, stats)


def kernel(en_1024, en_db, es_db, spanish_idx, W1, b1, g1, bn1, W2, b2, g2, bn2, W3, b3):
    qn = _p0(en_1024)
    sims, stats = _p1(qn, en_db)
    lo = _p2(sims, stats)
    cand = _p3(sims, jnp.asarray(lo[:, 0]))
    p_sel = (jnp.arange(D)[:, None] == spanish_idx[None, :]).astype(jnp.bfloat16)
    p_bf = jnp.pad(p_sel, ((0, 0), (0, 128 - ES_DIM)))
    es_spb = _p2b(es_db, p_bf)
    t = _p4(cand)
    out = _p5(sims, es_spb, t, stats)
    es101 = out[:, :ES_DIM]
    delta = jnp.zeros((B, ES_DIM), jnp.float32)
    return (es101, es101, delta)


# speedup vs baseline: 1.0741x; 1.0741x over previous
import functools

import jax
import jax.numpy as jnp
from jax import lax
from jax.experimental import pallas as pl
from jax.experimental.pallas import tpu as pltpu
from jax.experimental.pallas import tpu_sc as plsc

B, N, D = 1024, 100000, 1024
ES_DIM = 101
TOP_K = 90
TEMP = 0.04
BN = 2048
NBLK = 49
NP = NBLK * BN
CAP = 2048
CAPV = CAP // 16
NVREG = NP // 16
NW = 32
QPW = B // NW
NLEV = 8
NEG = float("-inf")


def _qnorm_body(x_ref, o_ref):
    x = x_ref[:]
    ss = jnp.sum(x * x, axis=1, keepdims=True)
    n = jnp.maximum(jnp.sqrt(ss), 1e-12)
    o_ref[:] = (x / n).astype(jnp.bfloat16)


def _p0(en_1024):
    return pl.pallas_call(
        _qnorm_body,
        out_shape=jax.ShapeDtypeStruct((B, D), jnp.bfloat16),
    )(en_1024)


def _sims_body(qn_ref, db_ref, sims_ref, stats_ref, acc_ref):
    j = pl.program_id(0)
    db = db_ref[:]
    ss = jnp.sum(db * db, axis=1, keepdims=True)
    dbn = (db / jnp.maximum(jnp.sqrt(ss), 1e-12)).astype(jnp.bfloat16)
    s = jax.lax.dot_general(qn_ref[:], dbn, (((1,), (1,)), ((), ())),
                            preferred_element_type=jnp.float32)
    col = j * BN + jax.lax.broadcasted_iota(jnp.int32, (B, BN), 1)
    valid = col < N
    s_m = jnp.where(valid, s, NEG)
    sims_ref[:] = s_m
    bmax = jnp.max(s_m, axis=1, keepdims=True)
    bsum = jnp.sum(jnp.where(valid, s, 0.0), axis=1, keepdims=True)
    bmin = jnp.min(jnp.where(valid, s, jnp.inf), axis=1, keepdims=True)
    first = j == 0
    a = acc_ref[:]
    nmax = jnp.where(first, bmax, jnp.maximum(a[:, 0:1], bmax))
    nsum = jnp.where(first, bsum, a[:, 1:2] + bsum)
    nmin = jnp.where(first, bmin, jnp.minimum(a[:, 2:3], bmin))
    acc_ref[:] = jnp.concatenate(
        [nmax, nsum, nmin, jnp.zeros((B, 5), jnp.float32)], axis=1)

    @pl.when(j == NBLK - 1)
    def _():
        stats_ref[:] = acc_ref[:]


def _p1(qn, en_db):
    return pl.pallas_call(
        _sims_body,
        grid=(NBLK,),
        in_specs=[
            pl.BlockSpec((B, D), lambda j: (0, 0)),
            pl.BlockSpec((BN, D), lambda j: (j, 0)),
        ],
        out_specs=[
            pl.BlockSpec((B, BN), lambda j: (0, j)),
            pl.BlockSpec((B, 8), lambda j: (0, 0)),
        ],
        out_shape=[
            jax.ShapeDtypeStruct((B, NP), jnp.float32),
            jax.ShapeDtypeStruct((B, 8), jnp.float32),
        ],
        scratch_shapes=[pltpu.VMEM((B, 8), jnp.float32)],
    )(qn, en_db)


def _count_body(sims_ref, stats_ref, lo_ref, cnt_ref):
    j = pl.program_id(0)
    st = stats_ref[:]
    mx = st[:, 0:1]
    mean = st[:, 1:2] * (1.0 / N)
    mn = st[:, 2:3]
    lane = jax.lax.broadcasted_iota(jnp.int32, (B, NLEV), 1)
    frac = (lane.astype(jnp.float32) + 1.0) * (1.0 / (NLEV - 1.0))
    levels = mx - (mx - mean) * frac
    levels = jnp.where(lane == NLEV - 1, mn, levels)
    s = sims_ref[:]
    parts = []
    for l in range(NLEV):
        m = (s >= levels[:, l:l + 1]).astype(jnp.float32)
        parts.append(jnp.sum(m, axis=1, keepdims=True))
    blk = jnp.concatenate(parts, axis=1)
    cnt_ref[:] = jnp.where(j == 0, blk, cnt_ref[:] + blk)

    @pl.when(j == NBLK - 1)
    def _():
        cnt = cnt_ref[:]
        lo = jnp.max(jnp.where(cnt >= TOP_K, levels, NEG), axis=1, keepdims=True)
        lo_ref[:] = jnp.broadcast_to(lo, (B, 8))


def _p2(sims, stats):
    return pl.pallas_call(
        _count_body,
        grid=(NBLK,),
        in_specs=[
            pl.BlockSpec((B, BN), lambda j: (0, j)),
            pl.BlockSpec((B, 8), lambda j: (0, 0)),
        ],
        out_specs=pl.BlockSpec((B, 8), lambda j: (0, 0)),
        out_shape=jax.ShapeDtypeStruct((B, 8), jnp.float32),
        scratch_shapes=[pltpu.VMEM((B, NLEV), jnp.float32)],
    )(sims, stats)


NCH = 4
CH = NP // NCH
CHV = CH // 16


def _sc_compact_body(sims_hbm, lo_hbm, cand_hbm, buf0, buf1, cand_v, lo_v,
                     sem0, sem1):
    wid = lax.axis_index("s") * 2 + lax.axis_index("c")
    base = wid * QPW
    pltpu.sync_copy(lo_hbm.at[pl.ds(base, QPW)], lo_v.at[pl.ds(0, QPW)])
    zeros16 = jnp.zeros((16,), jnp.int32)
    neg16 = jnp.full((16,), NEG, jnp.float32)
    bufs = (buf0, buf1)
    sems = (sem0, sem1)

    def per_query(k, carry):
        q = base + k
        lo_vec = jnp.zeros((16,), jnp.float32) + lo_v[pl.ds(k, 16)][0]

        @plsc.parallel_loop(0, CAPV, unroll=8)
        def clear(s):
            cand_v[pl.ds(s * 16, 16)] = neg16

        copies = [pltpu.async_copy(sims_hbm.at[q, pl.ds(0, CH)], bufs[0], sems[0])]
        off = jnp.int32(0)
        for ci in range(NCH):
            copies[ci].wait()
            if ci + 1 < NCH:
                copies.append(pltpu.async_copy(
                    sims_hbm.at[q, pl.ds((ci + 1) * CH, CH)],
                    bufs[(ci + 1) % 2], sems[(ci + 1) % 2]))
            buf = bufs[ci % 2]

            def pass1(i, off_s, buf=buf):
                v = buf[pl.ds(i * 16, 16)]
                m = v >= lo_vec
                ones = jnp.where(m, jnp.int32(1), jnp.int32(0))
                c = plsc.cumsum(ones)
                dest = jnp.minimum(zeros16 + off_s + c - 1, CAP - 1)
                plsc.store_scatter(cand_v, [dest], v, mask=m)
                return off_s + c[15]

            off = plsc.parallel_loop(0, CHV, carry=off, unroll=8)(pass1)

        pltpu.sync_copy(cand_v, cand_hbm.at[q])
        return carry

    lax.fori_loop(0, QPW, per_query, 0)


def _p3(sims, lo1):
    f = functools.partial(
        pl.kernel,
        out_type=jax.ShapeDtypeStruct((B, CAP), jnp.float32),
        mesh=plsc.VectorSubcoreMesh(core_axis_name="c", subcore_axis_name="s"),
        compiler_params=pltpu.CompilerParams(needs_layout_passes=False),
        scratch_types=[
            pltpu.VMEM((CH,), jnp.float32),
            pltpu.VMEM((CH,), jnp.float32),
            pltpu.VMEM((CAP,), jnp.float32),
            pltpu.VMEM((128,), jnp.float32),
            pltpu.SemaphoreType.DMA,
            pltpu.SemaphoreType.DMA,
        ],
    )(_sc_compact_body)
    return f(sims, lo1)


def _bisect_body(cand_ref, t_ref):
    c = cand_ref[:]
    bu = jax.lax.bitcast_convert_type(c, jnp.uint32)
    key = jnp.where(c >= 0, bu ^ jnp.uint32(0x80000000), ~bu)
    rows = c.shape[0]
    t = jnp.zeros((rows, 1), jnp.uint32)
    for bit in range(31, -1, -1):
        trial = t + jnp.uint32(1 << bit)
        n_ge = jnp.sum((key >= trial).astype(jnp.int32), axis=1, keepdims=True)
        t = jnp.where(n_ge >= TOP_K, trial, t)
    fb = jnp.where(t >= jnp.uint32(0x80000000), t ^ jnp.uint32(0x80000000), ~t)
    v90 = jax.lax.bitcast_convert_type(fb, jnp.float32)
    t_ref[:] = jnp.broadcast_to(v90, (rows, 8))


def _p4(cand):
    rb = 512
    return pl.pallas_call(
        _bisect_body,
        grid=(B // rb,),
        in_specs=[pl.BlockSpec((rb, CAP), lambda r: (r, 0))],
        out_specs=pl.BlockSpec((rb, 8), lambda r: (r, 0)),
        out_shape=jax.ShapeDtypeStruct((B, 8), jnp.float32),
    )(cand)


def _selmat_body(es_ref, p_ref, out_ref):
    j = pl.program_id(0)
    es_sp = jax.lax.dot_general(es_ref[:].astype(jnp.bfloat16), p_ref[:],
                                (((1,), (0,)), ((), ())),
                                preferred_element_type=jnp.float32)
    row = j * BN + jax.lax.broadcasted_iota(jnp.int32, (BN, 128), 0)
    colid = jax.lax.broadcasted_iota(jnp.int32, (BN, 128), 1)
    es_sp = jnp.where(colid == ES_DIM, 1.0, jnp.where(row < N, es_sp, 0.0))
    out_ref[:] = es_sp.astype(jnp.bfloat16)


def _p2b(es_db, p_bf):
    return pl.pallas_call(
        _selmat_body,
        grid=(NBLK,),
        in_specs=[
            pl.BlockSpec((BN, D), lambda j: (j, 0)),
            pl.BlockSpec((D, 128), lambda j: (0, 0)),
        ],
        out_specs=pl.BlockSpec((BN, 128), lambda j: (j, 0)),
        out_shape=jax.ShapeDtypeStruct((NP, 128), jnp.bfloat16),
    )(es_db, p_bf)


def _combine_body(sims_ref, esp_ref, t_ref, stats_ref, out_ref, acc_ref):
    j = pl.program_id(0)
    s = sims_ref[:]
    t = t_ref[:, 0:1]
    mx = stats_ref[:, 0:1]
    w = jnp.where(s >= t, jnp.exp((s - mx) * (1.0 / TEMP)), 0.0)
    contrib = jax.lax.dot_general(w.astype(jnp.bfloat16), esp_ref[:],
                                  (((1,), (0,)), ((), ())),
                                  preferred_element_type=jnp.float32)
    acc_ref[:] = jnp.where(j == 0, contrib, acc_ref[:] + contrib)

    @pl.when(j == NBLK - 1)
    def _():
        a = acc_ref[:]
        out_ref[:] = a / a[:, ES_DIM:ES_DIM + 1]


def _p5(sims, es_spb, t, stats):
    return pl.pallas_call(
        _combine_body,
        grid=(NBLK,),
        in_specs=[
            pl.BlockSpec((B, BN), lambda j: (0, j)),
            pl.BlockSpec((BN, 128), lambda j: (j, 0)),
            pl.BlockSpec((B, 8), lambda j: (0, 0)),
            pl.BlockSpec((B, 8), lambda j: (0, 0)),
        ],
        out_specs=pl.BlockSpec((B, 128), lambda j: (0, 0)),
        out_shape=jax.ShapeDtypeStruct((B, 128), jnp.float32),
        scratch_shapes=[pltpu.VMEM((B, 128), jnp.float32)],
    )(sims, es_spb, t, stats)


def kernel(en_1024, en_db, es_db, spanish_idx, W1, b1, g1, bn1, W2, b2, g2, bn2, W3, b3):
    qn = _p0(en_1024)
    sims, stats = _p1(qn, en_db)
    lo = _p2(sims, stats)
    cand = _p3(sims, jnp.asarray(lo[:, 0]))
    p_sel = (jnp.arange(D)[:, None] == spanish_idx[None, :]).astype(jnp.bfloat16)
    p_bf = jnp.pad(p_sel, ((0, 0), (0, 128 - ES_DIM)))
    es_spb = _p2b(es_db, p_bf)
    t = _p4(cand)
    out = _p5(sims, es_spb, t, stats)
    es101 = out[:, :ES_DIM]
    delta = jnp.zeros((B, ES_DIM), jnp.float32)
    return (es101, es101, delta)


# speedup vs baseline: 1.1176x; 1.0405x over previous
import functools

import jax
import jax.numpy as jnp
from jax import lax
from jax.experimental import pallas as pl
from jax.experimental.pallas import tpu as pltpu
from jax.experimental.pallas import tpu_sc as plsc

B, N, D = 1024, 100000, 1024
ES_DIM = 101
TOP_K = 90
TEMP = 0.04
BN = 2048
NBLK = 49
NP = NBLK * BN
CAP = 2048
CAPV = CAP // 16
NVREG = NP // 16
NW = 32
QPW = (B // 2) // NW
NLEV = 8
NEG = float("-inf")


def _qnorm_body(x_ref, o_ref):
    x = x_ref[:]
    ss = jnp.sum(x * x, axis=1, keepdims=True)
    n = jnp.maximum(jnp.sqrt(ss), 1e-12)
    o_ref[:] = (x / n).astype(jnp.bfloat16)


def _p0(en_1024):
    return pl.pallas_call(
        _qnorm_body,
        out_shape=jax.ShapeDtypeStruct((B, D), jnp.bfloat16),
    )(en_1024)


def _sims_body(qn_ref, db_ref, sims_ref, stats_ref, acc_ref):
    j = pl.program_id(0)
    db = db_ref[:]
    ss = jnp.sum(db * db, axis=1, keepdims=True)
    dbn = (db / jnp.maximum(jnp.sqrt(ss), 1e-12)).astype(jnp.bfloat16)
    s = jax.lax.dot_general(qn_ref[:], dbn, (((1,), (1,)), ((), ())),
                            preferred_element_type=jnp.float32)
    col = j * BN + jax.lax.broadcasted_iota(jnp.int32, (B, BN), 1)
    valid = col < N
    s_m = jnp.where(valid, s, NEG)
    sims_ref[:] = s_m
    bmax = jnp.max(s_m, axis=1, keepdims=True)
    bsum = jnp.sum(jnp.where(valid, s, 0.0), axis=1, keepdims=True)
    bmin = jnp.min(jnp.where(valid, s, jnp.inf), axis=1, keepdims=True)
    first = j == 0
    a = acc_ref[:]
    nmax = jnp.where(first, bmax, jnp.maximum(a[:, 0:1], bmax))
    nsum = jnp.where(first, bsum, a[:, 1:2] + bsum)
    nmin = jnp.where(first, bmin, jnp.minimum(a[:, 2:3], bmin))
    acc_ref[:] = jnp.concatenate(
        [nmax, nsum, nmin, jnp.zeros((B, 5), jnp.float32)], axis=1)

    @pl.when(j == NBLK - 1)
    def _():
        stats_ref[:] = acc_ref[:]


def _p1(qn, en_db):
    return pl.pallas_call(
        _sims_body,
        grid=(NBLK,),
        in_specs=[
            pl.BlockSpec((B, D), lambda j: (0, 0)),
            pl.BlockSpec((BN, D), lambda j: (j, 0)),
        ],
        out_specs=[
            pl.BlockSpec((B, BN), lambda j: (0, j)),
            pl.BlockSpec((B, 8), lambda j: (0, 0)),
        ],
        out_shape=[
            jax.ShapeDtypeStruct((B, NP), jnp.float32),
            jax.ShapeDtypeStruct((B, 8), jnp.float32),
        ],
        scratch_shapes=[pltpu.VMEM((B, 8), jnp.float32)],
    )(qn, en_db)


def _count_body(sims_ref, stats_ref, lo_ref, cnt_ref):
    j = pl.program_id(0)
    st = stats_ref[:]
    mx = st[:, 0:1]
    mean = st[:, 1:2] * (1.0 / N)
    mn = st[:, 2:3]
    lane = jax.lax.broadcasted_iota(jnp.int32, (B, NLEV), 1)
    frac = (lane.astype(jnp.float32) + 1.0) * (1.0 / (NLEV - 1.0))
    levels = mx - (mx - mean) * frac
    levels = jnp.where(lane == NLEV - 1, mn, levels)
    s = sims_ref[:]
    parts = []
    for l in range(NLEV):
        m = (s >= levels[:, l:l + 1]).astype(jnp.float32)
        parts.append(jnp.sum(m, axis=1, keepdims=True))
    blk = jnp.concatenate(parts, axis=1)
    cnt_ref[:] = jnp.where(j == 0, blk, cnt_ref[:] + blk)

    @pl.when(j == NBLK - 1)
    def _():
        cnt = cnt_ref[:]
        lo = jnp.max(jnp.where(cnt >= TOP_K, levels, NEG), axis=1, keepdims=True)
        lo_ref[:] = jnp.broadcast_to(lo, (B, 8))


def _p2(sims, stats):
    return pl.pallas_call(
        _count_body,
        grid=(NBLK,),
        in_specs=[
            pl.BlockSpec((B, BN), lambda j: (0, j)),
            pl.BlockSpec((B, 8), lambda j: (0, 0)),
        ],
        out_specs=pl.BlockSpec((B, 8), lambda j: (0, 0)),
        out_shape=jax.ShapeDtypeStruct((B, 8), jnp.float32),
        scratch_shapes=[pltpu.VMEM((B, NLEV), jnp.float32)],
    )(sims, stats)


NCH = 4
CH = NP // NCH
CHV = CH // 16


def _sc_compact_body(half, sims_hbm, lo_hbm, cand_hbm, buf0, buf1, cand_v, lo_v,
                     sem0, sem1):
    wid = lax.axis_index("s") * 2 + lax.axis_index("c")
    base = half * (B // 2) + wid * QPW
    pltpu.sync_copy(lo_hbm.at[pl.ds(base, QPW)], lo_v.at[pl.ds(0, QPW)])
    zeros16 = jnp.zeros((16,), jnp.int32)
    neg16 = jnp.full((16,), NEG, jnp.float32)
    bufs = (buf0, buf1)
    sems = (sem0, sem1)

    def per_query(k, carry):
        q = base + k
        lo_vec = jnp.zeros((16,), jnp.float32) + lo_v[pl.ds(k, 16)][0]

        @plsc.parallel_loop(0, CAPV, unroll=8)
        def clear(s):
            cand_v[pl.ds(s * 16, 16)] = neg16

        copies = [pltpu.async_copy(sims_hbm.at[q, pl.ds(0, CH)], bufs[0], sems[0])]
        off = jnp.int32(0)
        for ci in range(NCH):
            copies[ci].wait()
            if ci + 1 < NCH:
                copies.append(pltpu.async_copy(
                    sims_hbm.at[q, pl.ds((ci + 1) * CH, CH)],
                    bufs[(ci + 1) % 2], sems[(ci + 1) % 2]))
            buf = bufs[ci % 2]

            def pass1(i, off_s, buf=buf):
                v = buf[pl.ds(i * 16, 16)]
                m = v >= lo_vec
                ones = jnp.where(m, jnp.int32(1), jnp.int32(0))
                c = plsc.cumsum(ones)
                dest = jnp.minimum(zeros16 + off_s + c - 1, CAP - 1)
                plsc.store_scatter(cand_v, [dest], v, mask=m)
                return off_s + c[15]

            off = plsc.parallel_loop(0, CHV, carry=off, unroll=8)(pass1)

        pltpu.sync_copy(cand_v, cand_hbm.at[q - half * (B // 2)])
        return carry

    lax.fori_loop(0, QPW, per_query, 0)


def _p3(sims, lo1, half):
    f = functools.partial(
        pl.kernel,
        out_type=jax.ShapeDtypeStruct((B // 2, CAP), jnp.float32),
        mesh=plsc.VectorSubcoreMesh(core_axis_name="c", subcore_axis_name="s"),
        compiler_params=pltpu.CompilerParams(needs_layout_passes=False),
        scratch_types=[
            pltpu.VMEM((CH,), jnp.float32),
            pltpu.VMEM((CH,), jnp.float32),
            pltpu.VMEM((CAP,), jnp.float32),
            pltpu.VMEM((128,), jnp.float32),
            pltpu.SemaphoreType.DMA,
            pltpu.SemaphoreType.DMA,
        ],
    )(functools.partial(_sc_compact_body, half))
    return f(sims, lo1)


def _bisect_body(cand_ref, t_ref):
    c = cand_ref[:]
    bu = jax.lax.bitcast_convert_type(c, jnp.uint32)
    key = jnp.where(c >= 0, bu ^ jnp.uint32(0x80000000), ~bu)
    rows = c.shape[0]
    t = jnp.zeros((rows, 1), jnp.uint32)
    for bit in range(31, -1, -1):
        trial = t + jnp.uint32(1 << bit)
        n_ge = jnp.sum((key >= trial).astype(jnp.int32), axis=1, keepdims=True)
        t = jnp.where(n_ge >= TOP_K, trial, t)
    fb = jnp.where(t >= jnp.uint32(0x80000000), t ^ jnp.uint32(0x80000000), ~t)
    v90 = jax.lax.bitcast_convert_type(fb, jnp.float32)
    t_ref[:] = jnp.broadcast_to(v90, (rows, 8))


def _p4(cand):
    rb = 512
    return pl.pallas_call(
        _bisect_body,
        grid=(1,),
        in_specs=[pl.BlockSpec((rb, CAP), lambda r: (0, 0))],
        out_specs=pl.BlockSpec((rb, 8), lambda r: (0, 0)),
        out_shape=jax.ShapeDtypeStruct((B // 2, 8), jnp.float32),
    )(cand)


def _selmat_body(es_ref, p_ref, out_ref):
    j = pl.program_id(0)
    es_sp = jax.lax.dot_general(es_ref[:].astype(jnp.bfloat16), p_ref[:],
                                (((1,), (0,)), ((), ())),
                                preferred_element_type=jnp.float32)
    row = j * BN + jax.lax.broadcasted_iota(jnp.int32, (BN, 128), 0)
    colid = jax.lax.broadcasted_iota(jnp.int32, (BN, 128), 1)
    es_sp = jnp.where(colid == ES_DIM, 1.0, jnp.where(row < N, es_sp, 0.0))
    out_ref[:] = es_sp.astype(jnp.bfloat16)


def _p2b(es_db, p_bf):
    return pl.pallas_call(
        _selmat_body,
        grid=(NBLK,),
        in_specs=[
            pl.BlockSpec((BN, D), lambda j: (j, 0)),
            pl.BlockSpec((D, 128), lambda j: (0, 0)),
        ],
        out_specs=pl.BlockSpec((BN, 128), lambda j: (j, 0)),
        out_shape=jax.ShapeDtypeStruct((NP, 128), jnp.bfloat16),
    )(es_db, p_bf)


def _combine_body(sims_ref, esp_ref, t_ref, stats_ref, out_ref, acc_ref):
    j = pl.program_id(0)
    s = sims_ref[:]
    t = t_ref[:, 0:1]
    mx = stats_ref[:, 0:1]
    w = jnp.where(s >= t, jnp.exp((s - mx) * (1.0 / TEMP)), 0.0)
    contrib = jax.lax.dot_general(w.astype(jnp.bfloat16), esp_ref[:],
                                  (((1,), (0,)), ((), ())),
                                  preferred_element_type=jnp.float32)
    acc_ref[:] = jnp.where(j == 0, contrib, acc_ref[:] + contrib)

    @pl.when(j == NBLK - 1)
    def _():
        a = acc_ref[:]
        out_ref[:] = a / a[:, ES_DIM:ES_DIM + 1]


def _p5(sims, es_spb, t, stats_h, half):
    hb = B // 2
    return pl.pallas_call(
        _combine_body,
        grid=(NBLK,),
        in_specs=[
            pl.BlockSpec((hb, BN), lambda j, half=half: (half, j)),
            pl.BlockSpec((BN, 128), lambda j: (j, 0)),
            pl.BlockSpec((hb, 8), lambda j: (0, 0)),
            pl.BlockSpec((hb, 8), lambda j: (0, 0)),
        ],
        out_specs=pl.BlockSpec((hb, 128), lambda j: (0, 0)),
        out_shape=jax.ShapeDtypeStruct((hb, 128), jnp.float32),
        scratch_shapes=[pltpu.VMEM((hb, 128), jnp.float32)],
    )(sims, es_spb, t, stats_h)


def kernel(en_1024, en_db, es_db, spanish_idx, W1, b1, g1, bn1, W2, b2, g2, bn2, W3, b3):
    qn = _p0(en_1024)
    sims, stats = _p1(qn, en_db)
    lo = _p2(sims, stats)
    lo1 = jnp.asarray(lo[:, 0])
    hb = B // 2
    cand_a = _p3(sims, lo1, 0)
    p_sel = (jnp.arange(D)[:, None] == spanish_idx[None, :]).astype(jnp.bfloat16)
    p_bf = jnp.pad(p_sel, ((0, 0), (0, 128 - ES_DIM)))
    es_spb = _p2b(es_db, p_bf)
    cand_b = _p3(sims, lo1, 1)
    t_a = _p4(cand_a)
    out_a = _p5(sims, es_spb, t_a, stats[:hb], 0)
    t_b = _p4(cand_b)
    out_b = _p5(sims, es_spb, t_b, stats[hb:], 1)
    out = jnp.concatenate([out_a, out_b], axis=0)
    es101 = out[:, :ES_DIM]
    delta = jnp.zeros((B, ES_DIM), jnp.float32)
    return (es101, es101, delta)
